# baseline (speedup 1.0000x reference)
import jax
import jax.numpy as jnp
from jax import lax
from jax.experimental import pallas as pl
from jax.experimental.pallas import tpu as pltpu

D = 128
H = 64
CHUNK = 2048


def _body(idxa_ref, idxb_ref, ind_ref, wmT_ref, waT_ref, bm_ref, ba_ref,
          x_ref, out_ref,
          in0, in1, ob0, ob1, isem0, isem1, osem0, osem1):
    f32 = jnp.float32
    nch = x_ref.shape[0] // CHUNK
    inbufs = (in0, in1)
    obufs = (ob0, ob1)
    isems = (isem0, isem1)
    osems = (osem0, osem1)

    def in_copy(i):
        return pltpu.make_async_copy(
            x_ref.at[pl.ds(i * CHUNK, CHUNK), :], inbufs[i % 2], isems[i % 2])

    def out_copy(i):
        return pltpu.make_async_copy(
            obufs[i % 2], out_ref.at[pl.ds(i * CHUNK, CHUNK), :], osems[i % 2])

    in_copy(0).start()

    iota_dh = lax.broadcasted_iota(jnp.int32, (D, H), 0)
    ga = (iota_dh == idxa_ref[...]).astype(f32)
    gb = (iota_dh == idxb_ref[...]).astype(f32)
    iota_hd = lax.broadcasted_iota(jnp.int32, (H, D), 1)
    s = (iota_hd == ind_ref[...]).astype(f32)
    wm_full = jnp.dot(gb, wmT_ref[...], preferred_element_type=f32)
    wa_full = jnp.dot(gb, waT_ref[...], preferred_element_type=f32)
    keep = 1.0 - jnp.sum(s, axis=0, keepdims=True)
    iota_r = lax.broadcasted_iota(jnp.int32, (D, D), 0)
    iota_c = lax.broadcasted_iota(jnp.int32, (D, D), 1)
    Wm_f = jnp.dot(wm_full, s, preferred_element_type=f32)
    Ga_f = jnp.dot(ga, s, preferred_element_type=f32)
    M = jnp.where(iota_r == iota_c, keep, 0.0) \
        + jnp.dot(wa_full, s, preferred_element_type=f32)
    bm_f = jnp.dot(bm_ref[...], s, preferred_element_type=f32)
    ba_f = jnp.dot(ba_ref[...], s, preferred_element_type=f32)

    for i in range(nch):
        if i + 1 < nch:
            in_copy(i + 1).start()
        in_copy(i).wait()
        x = inbufs[i % 2][...]
        mul_f = jnp.dot(x, Wm_f, preferred_element_type=f32) + bm_f
        am_f = jnp.dot(x, Ga_f, preferred_element_type=f32)
        base_f = jnp.dot(x, M, preferred_element_type=f32) + ba_f
        if i >= 2:
            out_copy(i - 2).wait()
        obufs[i % 2][...] = mul_f * am_f + base_f
        out_copy(i).start()

    out_copy(nch - 2).wait()
    out_copy(nch - 1).wait()


def kernel(x, W_mul, b_mul, W_add, b_add, indices, idx_A, idx_B):
    n = x.shape[0]
    idxa = idx_A.reshape(1, H).astype(jnp.int32)
    idxb = idx_B.reshape(1, H).astype(jnp.int32)
    ind = indices.reshape(H, 1).astype(jnp.int32)
    wmT = W_mul.T
    waT = W_add.T
    bm = b_mul.reshape(1, H)
    ba = b_add.reshape(1, H)

    vmem = pl.BlockSpec(memory_space=pltpu.MemorySpace.VMEM)
    hbm = pl.BlockSpec(memory_space=pltpu.MemorySpace.HBM)
    return pl.pallas_call(
        _body,
        in_specs=[vmem, vmem, vmem, vmem, vmem, vmem, vmem, hbm],
        out_specs=hbm,
        out_shape=jax.ShapeDtypeStruct((n, D), jnp.float32),
        scratch_shapes=[
            pltpu.VMEM((CHUNK, D), jnp.float32),
            pltpu.VMEM((CHUNK, D), jnp.float32),
            pltpu.VMEM((CHUNK, D), jnp.float32),
            pltpu.VMEM((CHUNK, D), jnp.float32),
            pltpu.SemaphoreType.DMA,
            pltpu.SemaphoreType.DMA,
            pltpu.SemaphoreType.DMA,
            pltpu.SemaphoreType.DMA,
        ],
    )(idxa, idxb, ind, wmT, waT, bm, ba, x)


# speedup vs baseline: 1.2965x; 1.2965x over previous
import jax
import jax.numpy as jnp
from jax import lax
from jax.experimental import pallas as pl
from jax.experimental.pallas import tpu as pltpu

D = 128
H = 64
BLOCK = 8192


def _body(idxa_ref, idxb_ref, ind_ref, wmT_ref, waT_ref, bm_ref, ba_ref,
          x_ref, out_ref, k_ref, bmf_ref, baf_ref):
    f32 = jnp.float32

    @pl.when(pl.program_id(0) == 0)
    def _prep():
        iota_dh = lax.broadcasted_iota(jnp.int32, (D, H), 0)
        ga = (iota_dh == idxa_ref[...]).astype(f32)
        gb = (iota_dh == idxb_ref[...]).astype(f32)
        iota_hd = lax.broadcasted_iota(jnp.int32, (H, D), 1)
        s = (iota_hd == ind_ref[...]).astype(f32)
        wm_full = jnp.dot(gb, wmT_ref[...], preferred_element_type=f32)
        wa_full = jnp.dot(gb, waT_ref[...], preferred_element_type=f32)
        keep = 1.0 - jnp.sum(s, axis=0, keepdims=True)
        iota_r = lax.broadcasted_iota(jnp.int32, (D, D), 0)
        iota_c = lax.broadcasted_iota(jnp.int32, (D, D), 1)
        Wm_f = jnp.dot(wm_full, s, preferred_element_type=f32)
        Ga_f = jnp.dot(ga, s, preferred_element_type=f32)
        M = jnp.where(iota_r == iota_c, keep, 0.0) \
            + jnp.dot(wa_full, s, preferred_element_type=f32)
        k_ref[...] = jnp.concatenate([Wm_f, Ga_f, M], axis=1)
        bmf_ref[...] = jnp.dot(bm_ref[...], s, preferred_element_type=f32)
        baf_ref[...] = jnp.dot(ba_ref[...], s, preferred_element_type=f32)

    x = x_ref[...]
    acc = jnp.dot(x, k_ref[...], preferred_element_type=f32)
    out_ref[...] = (acc[:, :D] + bmf_ref[...]) * acc[:, D:2 * D] \
        + acc[:, 2 * D:] + baf_ref[...]


def kernel(x, W_mul, b_mul, W_add, b_add, indices, idx_A, idx_B):
    n = x.shape[0]
    grid = n // BLOCK
    idxa = idx_A.reshape(1, H).astype(jnp.int32)
    idxb = idx_B.reshape(1, H).astype(jnp.int32)
    ind = indices.reshape(H, 1).astype(jnp.int32)
    wmT = W_mul.T
    waT = W_add.T
    bm = b_mul.reshape(1, H)
    ba = b_add.reshape(1, H)

    rep = lambda shape: pl.BlockSpec(shape, lambda i: (0, 0))
    return pl.pallas_call(
        _body,
        grid=(grid,),
        in_specs=[
            rep((1, H)),
            rep((1, H)),
            rep((H, 1)),
            rep((H, H)),
            rep((H, H)),
            rep((1, H)),
            rep((1, H)),
            pl.BlockSpec((BLOCK, D), lambda i: (i, 0)),
        ],
        out_specs=pl.BlockSpec((BLOCK, D), lambda i: (i, 0)),
        out_shape=jax.ShapeDtypeStruct((n, D), jnp.float32),
        scratch_shapes=[
            pltpu.VMEM((D, 3 * D), jnp.float32),
            pltpu.VMEM((1, D), jnp.float32),
            pltpu.VMEM((1, D), jnp.float32),
        ],
    )(idxa, idxb, ind, wmT, waT, bm, ba, x)


# speedup vs baseline: 1.9580x; 1.5102x over previous
import jax
import jax.numpy as jnp
from jax import lax
from jax.experimental import pallas as pl
from jax.experimental.pallas import tpu as pltpu

D = 128
H = 64
BLOCK = 8192


def _body(idxa_ref, idxb_ref, ind_ref, wm_ref, wa_ref, bm_ref, ba_ref,
          x_ref, out_ref, k_ref, bmf_ref, baf_ref):
    f32 = jnp.float32

    @pl.when(pl.program_id(0) == 0)
    def _prep():
        idxa = idxa_ref[...].reshape(1, H)
        idxb = idxb_ref[...].reshape(1, H)
        ind = ind_ref[...].reshape(1, H)
        bm = bm_ref[...].reshape(1, H)
        ba = ba_ref[...].reshape(1, H)
        iota_dh = lax.broadcasted_iota(jnp.int32, (D, H), 0)
        ga = (iota_dh == idxa).astype(f32)
        gb = (iota_dh == idxb).astype(f32)
        s = (iota_dh == ind).astype(f32).T
        t_dims = (((1,), (1,)), ((), ()))
        wm_full = lax.dot_general(gb, wm_ref[...], t_dims, preferred_element_type=f32)
        wa_full = lax.dot_general(gb, wa_ref[...], t_dims, preferred_element_type=f32)
        keep = 1.0 - jnp.sum(s, axis=0, keepdims=True)
        iota_r = lax.broadcasted_iota(jnp.int32, (D, D), 0)
        iota_c = lax.broadcasted_iota(jnp.int32, (D, D), 1)
        Wm_f = jnp.dot(wm_full, s, preferred_element_type=f32)
        Ga_f = jnp.dot(ga, s, preferred_element_type=f32)
        M = jnp.where(iota_r == iota_c, keep, 0.0) \
            + jnp.dot(wa_full, s, preferred_element_type=f32)
        k_ref[...] = jnp.concatenate([Wm_f, Ga_f, M], axis=1)
        bmf_ref[...] = jnp.dot(bm, s, preferred_element_type=f32)
        baf_ref[...] = jnp.dot(ba, s, preferred_element_type=f32)

    x = x_ref[...]
    acc = jnp.dot(x, k_ref[...], preferred_element_type=f32)
    out_ref[...] = (acc[:, :D] + bmf_ref[...]) * acc[:, D:2 * D] \
        + acc[:, 2 * D:] + baf_ref[...]


def kernel(x, W_mul, b_mul, W_add, b_add, indices, idx_A, idx_B):
    n = x.shape[0]
    grid = n // BLOCK

    vec = pl.BlockSpec((H,), lambda i: (0,))
    mat = pl.BlockSpec((H, H), lambda i: (0, 0))
    return pl.pallas_call(
        _body,
        grid=(grid,),
        in_specs=[
            vec,
            vec,
            vec,
            mat,
            mat,
            vec,
            vec,
            pl.BlockSpec((BLOCK, D), lambda i: (i, 0)),
        ],
        out_specs=pl.BlockSpec((BLOCK, D), lambda i: (i, 0)),
        out_shape=jax.ShapeDtypeStruct((n, D), jnp.float32),
        scratch_shapes=[
            pltpu.VMEM((D, 3 * D), jnp.float32),
            pltpu.VMEM((1, D), jnp.float32),
            pltpu.VMEM((1, D), jnp.float32),
        ],
    )(idx_A, idx_B, indices, W_mul, W_add, b_mul, b_add, x)
